# baseline (speedup 1.0000x reference)
import functools
import operator

import jax
import jax.numpy as jnp
from jax import lax
from jax.experimental import pallas as pl
from jax.experimental.pallas import tpu as pltpu
from jax.experimental.pallas import tpu_sc as plsc

M = 1024
D = 8192
DP = D // 32
B = 128

NC = 2
NS = 16
NW = 16
BPW = B // NW

_MESH = plsc.VectorSubcoreMesh(core_axis_name="c", subcore_axis_name="s")


@functools.partial(
    pl.kernel,
    out_type=jax.ShapeDtypeStruct((B, DP), jnp.uint32),
    mesh=_MESH,
    scratch_types=[
        pltpu.VMEM((BPW,), jnp.int32),
        pltpu.VMEM((BPW, DP), jnp.uint32),
        pltpu.SemaphoreType.DMA,
    ],
)
def _gather_rows(masks_hbm, idx_hbm, out_hbm, idx_v, rows_v, sem):
    wid = lax.axis_index("s") * NC + lax.axis_index("c")

    @pl.when(wid < NW)
    def _():
        base = wid * BPW
        pltpu.sync_copy(idx_hbm.at[pl.ds(base, BPW)], idx_v)
        pltpu.async_copy(masks_hbm.at[idx_v], rows_v, sem).wait()
        pltpu.sync_copy(rows_v, out_hbm.at[pl.ds(base, BPW)])


def kernel(masks, idx):
    packed = functools.reduce(operator.or_, [
        jnp.where(masks[:, k * DP:(k + 1) * DP],
                  jnp.uint32(1) << k, jnp.uint32(0))
        for k in range(32)
    ])
    out32 = _gather_rows(packed, idx)
    return jnp.concatenate(
        [(out32 & (jnp.uint32(1) << k)) != 0 for k in range(32)], axis=1)


# speedup vs baseline: 1.0204x; 1.0204x over previous
import functools
import operator

import jax
import jax.numpy as jnp
from jax import lax
from jax.experimental import pallas as pl
from jax.experimental.pallas import tpu as pltpu
from jax.experimental.pallas import tpu_sc as plsc

M = 1024
D = 8192
DP = D // 32
B = 128

NC = 2
NS = 16
NW = 16
BPW = B // NW

_MESH = plsc.VectorSubcoreMesh(core_axis_name="c", subcore_axis_name="s")


@functools.partial(
    pl.kernel,
    out_type=jax.ShapeDtypeStruct((B, DP), jnp.uint32),
    mesh=_MESH,
    scratch_types=[
        pltpu.VMEM((BPW,), jnp.int32),
        pltpu.VMEM((BPW, DP), jnp.uint32),
        pltpu.SemaphoreType.DMA,
    ],
)
def _gather_rows(masks_hbm, idx_hbm, out_hbm, idx_v, rows_v, sem):
    wid = lax.axis_index("s") * NC + lax.axis_index("c")

    @pl.when(wid < NW)
    def _():
        base = wid * BPW
        pltpu.sync_copy(idx_hbm.at[pl.ds(base, BPW)], idx_v)
        pltpu.async_copy(masks_hbm.at[idx_v], rows_v, sem).wait()
        pltpu.sync_copy(rows_v, out_hbm.at[pl.ds(base, BPW)])


def kernel(masks, idx):
    packed = functools.reduce(operator.or_, [
        masks[:, k * DP:(k + 1) * DP].astype(jnp.uint32) << k
        for k in range(32)
    ])
    out32 = _gather_rows(packed, idx)
    return jnp.concatenate(
        [(out32 & (jnp.uint32(1) << k)) != 0 for k in range(32)], axis=1)


# speedup vs baseline: 1.3943x; 1.3664x over previous
import functools
import operator

import jax
import jax.numpy as jnp
from jax import lax
from jax.experimental import pallas as pl
from jax.experimental.pallas import tpu as pltpu
from jax.experimental.pallas import tpu_sc as plsc

M = 1024
D = 8192
DP = D // 32
B = 128

NC = 2
NS = 16
NW = 16
BPW = B // NW

_MESH = plsc.VectorSubcoreMesh(core_axis_name="c", subcore_axis_name="s")


@functools.partial(
    pl.kernel,
    out_type=jax.ShapeDtypeStruct((B, DP), jnp.uint32),
    mesh=_MESH,
    scratch_types=[
        pltpu.VMEM((BPW,), jnp.int32),
        pltpu.VMEM((BPW, DP), jnp.uint32),
        pltpu.SemaphoreType.DMA,
    ],
)
def _gather_rows(masks_hbm, idx_hbm, out_hbm, idx_v, rows_v, sem):
    wid = lax.axis_index("s") * NC + lax.axis_index("c")

    @pl.when(wid < NW)
    def _():
        base = wid * BPW
        pltpu.sync_copy(idx_hbm.at[pl.ds(base, BPW)], idx_v)
        pltpu.async_copy(masks_hbm.at[idx_v], rows_v, sem).wait()
        pltpu.sync_copy(rows_v, out_hbm.at[pl.ds(base, BPW)])


def kernel(masks, idx):
    packed = functools.reduce(operator.or_, [
        masks[:, k * DP:(k + 1) * DP].astype(jnp.uint32) << k
        for k in range(32)
    ])
    out32 = _gather_rows(packed, idx)
    return jnp.concatenate(
        [out32 & (jnp.uint32(1) << k) for k in range(32)], axis=1) != 0


# speedup vs baseline: 1.8173x; 1.3034x over previous
import functools
import operator

import jax
import jax.numpy as jnp
from jax import lax
from jax.experimental import pallas as pl
from jax.experimental.pallas import tpu as pltpu
from jax.experimental.pallas import tpu_sc as plsc

M = 1024
D = 8192
PF = 8
DP = D // PF
B = 128

NC = 2
NS = 16
NW = 16
BPW = B // NW

_MESH = plsc.VectorSubcoreMesh(core_axis_name="c", subcore_axis_name="s")


@functools.partial(
    pl.kernel,
    out_type=jax.ShapeDtypeStruct((B, DP), jnp.uint32),
    mesh=_MESH,
    scratch_types=[
        pltpu.VMEM((BPW,), jnp.int32),
        pltpu.VMEM((BPW, DP), jnp.uint32),
        pltpu.SemaphoreType.DMA,
    ],
)
def _gather_rows(masks_hbm, idx_hbm, out_hbm, idx_v, rows_v, sem):
    wid = lax.axis_index("s") * NC + lax.axis_index("c")

    @pl.when(wid < NW)
    def _():
        base = wid * BPW
        pltpu.sync_copy(idx_hbm.at[pl.ds(base, BPW)], idx_v)
        pltpu.async_copy(masks_hbm.at[idx_v], rows_v, sem).wait()
        pltpu.sync_copy(rows_v, out_hbm.at[pl.ds(base, BPW)])


def kernel(masks, idx):
    packed = functools.reduce(operator.or_, [
        masks[:, k * DP:(k + 1) * DP].astype(jnp.uint32) << k
        for k in range(PF)
    ])
    out32 = _gather_rows(packed, idx)
    return jnp.concatenate(
        [out32 & (jnp.uint32(1) << k) for k in range(PF)], axis=1) != 0


# speedup vs baseline: 1.9163x; 1.0545x over previous
import functools
import operator

import jax
import jax.numpy as jnp
from jax import lax
from jax.experimental import pallas as pl
from jax.experimental.pallas import tpu as pltpu
from jax.experimental.pallas import tpu_sc as plsc

M = 1024
D = 8192
PF = 4
DP = D // PF
B = 128

NC = 2
NS = 16
NW = 16
BPW = B // NW

_MESH = plsc.VectorSubcoreMesh(core_axis_name="c", subcore_axis_name="s")


@functools.partial(
    pl.kernel,
    out_type=jax.ShapeDtypeStruct((B, DP), jnp.uint32),
    mesh=_MESH,
    scratch_types=[
        pltpu.VMEM((BPW,), jnp.int32),
        pltpu.VMEM((BPW, DP), jnp.uint32),
        pltpu.SemaphoreType.DMA,
    ],
)
def _gather_rows(masks_hbm, idx_hbm, out_hbm, idx_v, rows_v, sem):
    wid = lax.axis_index("s") * NC + lax.axis_index("c")

    @pl.when(wid < NW)
    def _():
        base = wid * BPW
        pltpu.sync_copy(idx_hbm.at[pl.ds(base, BPW)], idx_v)
        pltpu.async_copy(masks_hbm.at[idx_v], rows_v, sem).wait()
        pltpu.sync_copy(rows_v, out_hbm.at[pl.ds(base, BPW)])


def kernel(masks, idx):
    packed = functools.reduce(operator.or_, [
        masks[:, k * DP:(k + 1) * DP].astype(jnp.uint32) << k
        for k in range(PF)
    ])
    out32 = _gather_rows(packed, idx)
    return jnp.concatenate(
        [out32 & (jnp.uint32(1) << k) for k in range(PF)], axis=1) != 0
